# Initial kernel scaffold; baseline (speedup 1.0000x reference)
import jax
import jax.numpy as jnp
from jax.experimental import pallas as pl


def kernel(x, neighbors, gamma1, beta1, W1, b1, gamma2, beta2, W2, b2):
    raise NotImplementedError("write your pallas kernel here")


# speedup vs baseline: 3.1417x; 3.1417x over previous
import functools
import math

import jax
import jax.numpy as jnp
from jax.experimental import pallas as pl
from jax.experimental.pallas import tpu as pltpu

_B, _T, _N, _F, _K = 8, 16, 1024, 128, 4
_TF = 3 * _F
_P = 2
_EPS = 1e-5
_INV_SQRT3 = 1.0 / math.sqrt(3.0)
_INV_SQRT2 = 1.0 / math.sqrt(2.0)


def _stats_body(x_ref, out_ref, s_ref, ss_ref):
    i = pl.program_id(0)
    xb = x_ref[0]
    xr = xb.reshape(_N // 8, 8, _F)

    @pl.when(i == 0)
    def _():
        s_ref[...] = jnp.zeros_like(s_ref)
        ss_ref[...] = jnp.zeros_like(ss_ref)

    s_ref[...] += jnp.sum(xr, axis=0)
    ss_ref[...] += jnp.sum(xr * xr, axis=0)

    @pl.when(i == pl.num_programs(0) - 1)
    def _():
        out_ref[0:8, :] = s_ref[...]
        out_ref[8:16, :] = ss_ref[...]


def _moment_sums(x):
    return pl.pallas_call(
        _stats_body,
        grid=(_B * _T,),
        in_specs=[pl.BlockSpec((1, _N, _F), lambda i: (i, 0, 0))],
        out_specs=pl.BlockSpec((16, _F), lambda i: (0, 0)),
        out_shape=jax.ShapeDtypeStruct((16, _F), jnp.float32),
        scratch_shapes=[
            pltpu.VMEM((8, _F), jnp.float32),
            pltpu.VMEM((8, _F), jnp.float32),
        ],
        compiler_params=pltpu.CompilerParams(
            dimension_semantics=("arbitrary",)),
    )(x)


def _layer_body(add_identity, need_stats, *refs):
    if add_identity:
        (x_ref, prm_ref, bias_ref, w_ref, nbr_ref, idn_ref,
         out_ref, st_ref, S_ref, y_ref, acc_ref, py0_ref, s_ref,
         ss_ref) = refs
    else:
        (x_ref, prm_ref, bias_ref, w_ref, nbr_ref,
         out_ref, st_ref, S_ref, y_ref, acc_ref, py0_ref, s_ref,
         ss_ref) = refs
        idn_ref = None
    bp = pl.program_id(0)
    t = pl.program_id(1)

    @pl.when((bp == 0) & (t == 0))
    def _build():
        iota = jax.lax.broadcasted_iota(jnp.int32, (_N, _N), 1)
        for k in range(_K):
            col = nbr_ref[:, k:k + 1]
            S_ref[k] = (iota == col).astype(jnp.bfloat16)
        s_ref[...] = jnp.zeros_like(s_ref)
        ss_ref[...] = jnp.zeros_like(ss_ref)

    a = prm_ref[0:1, :]
    c = prm_ref[1:2, :]
    bias = bias_ref[0:1, :]

    @pl.when(t < _T)
    def _compute():
        xs = x_ref[...]
        v = xs[:, 0] * a + c
        v = 0.5 * v * (1.0 + jax.lax.erf(v * _INV_SQRT2))
        vb = v.astype(jnp.bfloat16)
        vcat = jnp.concatenate([vb[b] for b in range(_P)], axis=1)
        gk = [
            jax.lax.dot_general(
                S_ref[k], vcat, (((1,), (0,)), ((), ())),
                preferred_element_type=jnp.float32).astype(jnp.bfloat16)
            for k in range(_K)
        ]
        for b in range(_P):
            gb = jnp.concatenate(
                [g[:, b * _F:(b + 1) * _F] for g in gk], axis=1)
            yb = jax.lax.dot_general(
                gb, w_ref[...], (((1,), (0,)), ((), ())),
                preferred_element_type=jnp.float32)
            y_ref[b] = yb + bias

    def _emit(b, o):
        if add_identity:
            o = o + idn_ref[b, 0]
        out_ref[b, 0] = o
        if need_stats:
            orr = o.reshape(_N // 8, 8, _F)
            s_ref[...] += jnp.sum(orr, axis=0)
            ss_ref[...] += jnp.sum(orr * orr, axis=0)

    @pl.when(t == 0)
    def _t0():
        for b in range(_P):
            acc_ref[b] = y_ref[b, :, _F:2 * _F]
            py0_ref[b] = y_ref[b, :, 0:_F]

    @pl.when((t > 0) & (t < _T))
    def _mid():
        for b in range(_P):
            _emit(b, (acc_ref[b] + y_ref[b, :, 2 * _F:]) * _INV_SQRT3)
            acc_ref[b] = py0_ref[b] + y_ref[b, :, _F:2 * _F]
            py0_ref[b] = y_ref[b, :, 0:_F]

    @pl.when(t == _T)
    def _last():
        for b in range(_P):
            _emit(b, acc_ref[b] * _INV_SQRT3)

    if need_stats:
        @pl.when((bp == pl.num_programs(0) - 1) & (t == _T))
        def _st_out():
            st_ref[0:8, :] = s_ref[...]
            st_ref[8:16, :] = ss_ref[...]


def _layer(x, prm, bias, wbf, nbr, identity=None, need_stats=True):
    add_identity = identity is not None
    body = functools.partial(_layer_body, add_identity, need_stats)
    in_specs = [
        pl.BlockSpec((_P, 1, _N, _F),
                     lambda bp, t: (bp, jnp.minimum(t, _T - 1), 0, 0)),
        pl.BlockSpec((8, _F), lambda bp, t: (0, 0)),
        pl.BlockSpec((8, _TF), lambda bp, t: (0, 0)),
        pl.BlockSpec((_K * _F, _TF), lambda bp, t: (0, 0)),
        pl.BlockSpec((_N, _K), lambda bp, t: (0, 0)),
    ]
    args = [x, prm, bias, wbf, nbr]
    if add_identity:
        in_specs.append(
            pl.BlockSpec((_P, 1, _N, _F),
                         lambda bp, t: (bp, jnp.maximum(t - 1, 0), 0, 0)))
        args.append(identity)
    out_specs = [
        pl.BlockSpec((_P, 1, _N, _F),
                     lambda bp, t: (bp, jnp.maximum(t - 1, 0), 0, 0)),
        pl.BlockSpec((16, _F), lambda bp, t: (0, 0)),
    ]
    out_shape = [
        jax.ShapeDtypeStruct((_B, _T, _N, _F), jnp.float32),
        jax.ShapeDtypeStruct((16, _F), jnp.float32),
    ]
    h, st = pl.pallas_call(
        body,
        grid=(_B // _P, _T + 1),
        in_specs=in_specs,
        out_specs=out_specs,
        out_shape=out_shape,
        scratch_shapes=[
            pltpu.VMEM((_K, _N, _N), jnp.bfloat16),
            pltpu.VMEM((_P, _N, _TF), jnp.float32),
            pltpu.VMEM((_P, _N, _F), jnp.float32),
            pltpu.VMEM((_P, _N, _F), jnp.float32),
            pltpu.VMEM((8, _F), jnp.float32),
            pltpu.VMEM((8, _F), jnp.float32),
        ],
        compiler_params=pltpu.CompilerParams(
            dimension_semantics=("arbitrary", "arbitrary")),
    )(*args)
    return h, st


def _affine(st, gamma, beta):
    cnt = float(_B * _T * _N)
    s = jnp.sum(st[0:8, :], axis=0)
    ss = jnp.sum(st[8:16, :], axis=0)
    mean = s / cnt
    var = ss / cnt - mean * mean
    rstd = jax.lax.rsqrt(var + _EPS)
    a = gamma * rstd
    c = beta - mean * a
    prm = jnp.zeros((8, _F), jnp.float32).at[0].set(a).at[1].set(c)
    return prm


def _bias_rows(b):
    return jnp.zeros((8, _TF), jnp.float32).at[0].set(b)


def kernel(x, neighbors, gamma1, beta1, W1, b1, gamma2, beta2, W2, b2):
    nbr = neighbors.astype(jnp.int32)
    st0 = _moment_sums(x.reshape(_B * _T, _N, _F))
    prm1 = _affine(st0, gamma1, beta1)
    h1, st1 = _layer(x, prm1, _bias_rows(b1), W1.astype(jnp.bfloat16), nbr,
                     identity=None, need_stats=True)
    prm2 = _affine(st1, gamma2, beta2)
    out, _ = _layer(h1, prm2, _bias_rows(b2), W2.astype(jnp.bfloat16), nbr,
                    identity=x, need_stats=False)
    return out


# speedup vs baseline: 3.3508x; 1.0666x over previous
import functools
import math

import jax
import jax.numpy as jnp
from jax.experimental import pallas as pl
from jax.experimental.pallas import tpu as pltpu

_B, _T, _N, _F, _K = 8, 16, 1024, 128, 4
_TF = 3 * _F
_P = 4
_EPS = 1e-5
_INV_SQRT3 = 1.0 / math.sqrt(3.0)
_INV_SQRT2 = 1.0 / math.sqrt(2.0)


def _stats_body(x_ref, out_ref, s_ref, ss_ref):
    i = pl.program_id(0)
    xb = x_ref[0]
    xr = xb.reshape(_N // 8, 8, _F)

    @pl.when(i == 0)
    def _():
        s_ref[...] = jnp.zeros_like(s_ref)
        ss_ref[...] = jnp.zeros_like(ss_ref)

    s_ref[...] += jnp.sum(xr, axis=0)
    ss_ref[...] += jnp.sum(xr * xr, axis=0)

    @pl.when(i == pl.num_programs(0) - 1)
    def _():
        out_ref[0:8, :] = s_ref[...]
        out_ref[8:16, :] = ss_ref[...]


def _moment_sums(x):
    return pl.pallas_call(
        _stats_body,
        grid=(_B * _T,),
        in_specs=[pl.BlockSpec((1, _N, _F), lambda i: (i, 0, 0))],
        out_specs=pl.BlockSpec((16, _F), lambda i: (0, 0)),
        out_shape=jax.ShapeDtypeStruct((16, _F), jnp.float32),
        scratch_shapes=[
            pltpu.VMEM((8, _F), jnp.float32),
            pltpu.VMEM((8, _F), jnp.float32),
        ],
        compiler_params=pltpu.CompilerParams(
            dimension_semantics=("arbitrary",)),
    )(x)


def _onehot_body(nbr_ref, s_ref):
    iota = jax.lax.broadcasted_iota(jnp.int32, (_N, _N), 0)
    row = nbr_ref[0]
    s_ref[0] = (iota == row).astype(jnp.bfloat16)


def _onehot(nbr_t):
    return pl.pallas_call(
        _onehot_body,
        grid=(_K,),
        in_specs=[pl.BlockSpec((1, 1, _N), lambda k: (k, 0, 0))],
        out_specs=pl.BlockSpec((1, _N, _N), lambda k: (k, 0, 0)),
        out_shape=jax.ShapeDtypeStruct((_K, _N, _N), jnp.bfloat16),
        compiler_params=pltpu.CompilerParams(
            dimension_semantics=("arbitrary",)),
    )(nbr_t)


def _layer_body(add_identity, need_stats, *refs):
    if add_identity:
        (x_ref, prm_ref, bias_ref, w_ref, S_ref, idn_ref,
         out_ref, st_ref, acc_ref, py0_ref, s_ref, ss_ref) = refs
    else:
        (x_ref, prm_ref, bias_ref, w_ref, S_ref,
         out_ref, st_ref, acc_ref, py0_ref, s_ref, ss_ref) = refs
        idn_ref = None
    bp = pl.program_id(0)
    t = pl.program_id(1)

    @pl.when((bp == 0) & (t == 0))
    def _init():
        s_ref[...] = jnp.zeros_like(s_ref)
        ss_ref[...] = jnp.zeros_like(ss_ref)

    a = prm_ref[0:1, :]
    c = prm_ref[1:2, :]
    bias = bias_ref[0:1, :]

    xs = x_ref[...]
    v = xs[:, 0] * a + c
    v = 0.5 * v * (1.0 + jax.lax.erf(v * _INV_SQRT2))
    vb = v.astype(jnp.bfloat16)
    vcat = jnp.concatenate([vb[b] for b in range(_P)], axis=1)
    gk = [
        jax.lax.dot_general(
            S_ref[k], vcat, (((0,), (0,)), ((), ())),
            preferred_element_type=jnp.float32).astype(jnp.bfloat16)
        for k in range(_K)
    ]

    mT = (t < _T).astype(jnp.float32)
    m0 = (t > 0).astype(jnp.float32)

    for b in range(_P):
        gb = jnp.concatenate(
            [g[:, b * _F:(b + 1) * _F] for g in gk], axis=1)
        yb = jax.lax.dot_general(
            gb, w_ref[...], (((1,), (0,)), ((), ())),
            preferred_element_type=jnp.float32) + bias
        y0 = yb[:, 0:_F]
        y1 = yb[:, _F:2 * _F]
        y2 = yb[:, 2 * _F:]
        acc = jnp.where(t > 0, acc_ref[b], 0.0)
        py0 = jnp.where(t > 0, py0_ref[b], 0.0)
        o = (acc + mT * y2) * _INV_SQRT3
        if add_identity:
            o = o + idn_ref[b, 0]
        out_ref[b, 0] = o
        if need_stats:
            orr = o.reshape(_N // 8, 8, _F)
            s_ref[...] += m0 * jnp.sum(orr, axis=0)
            ss_ref[...] += m0 * jnp.sum(orr * orr, axis=0)
        acc_ref[b] = py0 + mT * y1
        py0_ref[b] = y0

    if need_stats:
        @pl.when((bp == pl.num_programs(0) - 1) & (t == _T))
        def _st_out():
            st_ref[0:8, :] = s_ref[...]
            st_ref[8:16, :] = ss_ref[...]


def _layer(x, prm, bias, wbf, S, identity=None, need_stats=True):
    add_identity = identity is not None
    body = functools.partial(_layer_body, add_identity, need_stats)
    in_specs = [
        pl.BlockSpec((_P, 1, _N, _F),
                     lambda bp, t: (bp, jnp.minimum(t, _T - 1), 0, 0)),
        pl.BlockSpec((8, _F), lambda bp, t: (0, 0)),
        pl.BlockSpec((8, _TF), lambda bp, t: (0, 0)),
        pl.BlockSpec((_K * _F, _TF), lambda bp, t: (0, 0)),
        pl.BlockSpec((_K, _N, _N), lambda bp, t: (0, 0, 0)),
    ]
    args = [x, prm, bias, wbf, S]
    if add_identity:
        in_specs.append(
            pl.BlockSpec((_P, 1, _N, _F),
                         lambda bp, t: (bp, jnp.maximum(t - 1, 0), 0, 0)))
        args.append(identity)
    out_specs = [
        pl.BlockSpec((_P, 1, _N, _F),
                     lambda bp, t: (bp, jnp.maximum(t - 1, 0), 0, 0)),
        pl.BlockSpec((16, _F), lambda bp, t: (0, 0)),
    ]
    out_shape = [
        jax.ShapeDtypeStruct((_B, _T, _N, _F), jnp.float32),
        jax.ShapeDtypeStruct((16, _F), jnp.float32),
    ]
    h, st = pl.pallas_call(
        body,
        grid=(_B // _P, _T + 1),
        in_specs=in_specs,
        out_specs=out_specs,
        out_shape=out_shape,
        scratch_shapes=[
            pltpu.VMEM((_P, _N, _F), jnp.float32),
            pltpu.VMEM((_P, _N, _F), jnp.float32),
            pltpu.VMEM((8, _F), jnp.float32),
            pltpu.VMEM((8, _F), jnp.float32),
        ],
        compiler_params=pltpu.CompilerParams(
            dimension_semantics=("arbitrary", "arbitrary")),
    )(*args)
    return h, st


def _affine(st, gamma, beta):
    cnt = float(_B * _T * _N)
    s = jnp.sum(st[0:8, :], axis=0)
    ss = jnp.sum(st[8:16, :], axis=0)
    mean = s / cnt
    var = ss / cnt - mean * mean
    rstd = jax.lax.rsqrt(var + _EPS)
    a = gamma * rstd
    c = beta - mean * a
    prm = jnp.zeros((8, _F), jnp.float32).at[0].set(a).at[1].set(c)
    return prm


def _bias_rows(b):
    return jnp.zeros((8, _TF), jnp.float32).at[0].set(b)


def kernel(x, neighbors, gamma1, beta1, W1, b1, gamma2, beta2, W2, b2):
    nbr_t = neighbors.astype(jnp.int32).T.reshape(_K, 1, _N)
    S = _onehot(nbr_t)
    st0 = _moment_sums(x.reshape(_B * _T, _N, _F))
    prm1 = _affine(st0, gamma1, beta1)
    h1, st1 = _layer(x, prm1, _bias_rows(b1), W1.astype(jnp.bfloat16), S,
                     identity=None, need_stats=True)
    prm2 = _affine(st1, gamma2, beta2)
    out, _ = _layer(h1, prm2, _bias_rows(b2), W2.astype(jnp.bfloat16), S,
                    identity=x, need_stats=False)
    return out


# speedup vs baseline: 3.5655x; 1.0641x over previous
import functools
import math

import jax
import jax.numpy as jnp
from jax.experimental import pallas as pl
from jax.experimental.pallas import tpu as pltpu

_B, _T, _N, _F, _K = 8, 16, 1024, 128, 4
_TF = 3 * _F
_P = 8
_EPS = 1e-5
_INV_SQRT3 = 1.0 / math.sqrt(3.0)
_INV_SQRT2 = 1.0 / math.sqrt(2.0)


def _stats_body(x_ref, out_ref, s_ref, ss_ref):
    i = pl.program_id(0)
    xb = x_ref[0]
    xr = xb.reshape(_N // 8, 8, _F)

    @pl.when(i == 0)
    def _():
        s_ref[...] = jnp.zeros_like(s_ref)
        ss_ref[...] = jnp.zeros_like(ss_ref)

    s_ref[...] += jnp.sum(xr, axis=0)
    ss_ref[...] += jnp.sum(xr * xr, axis=0)

    @pl.when(i == pl.num_programs(0) - 1)
    def _():
        out_ref[0:8, :] = s_ref[...]
        out_ref[8:16, :] = ss_ref[...]


def _moment_sums(x):
    return pl.pallas_call(
        _stats_body,
        grid=(_B * _T,),
        in_specs=[pl.BlockSpec((1, _N, _F), lambda i: (i, 0, 0))],
        out_specs=pl.BlockSpec((16, _F), lambda i: (0, 0)),
        out_shape=jax.ShapeDtypeStruct((16, _F), jnp.float32),
        scratch_shapes=[
            pltpu.VMEM((8, _F), jnp.float32),
            pltpu.VMEM((8, _F), jnp.float32),
        ],
        compiler_params=pltpu.CompilerParams(
            dimension_semantics=("arbitrary",)),
    )(x)


def _onehot_body(nbr_ref, s_ref):
    iota = jax.lax.broadcasted_iota(jnp.int32, (_N, _N), 0)
    row = nbr_ref[0]
    s_ref[...] = (iota == row).astype(jnp.bfloat16).T


def _onehot(nbr_t):
    return pl.pallas_call(
        _onehot_body,
        grid=(_K,),
        in_specs=[pl.BlockSpec((1, 1, _N), lambda k: (k, 0, 0))],
        out_specs=pl.BlockSpec((_N, _N), lambda k: (k, 0)),
        out_shape=jax.ShapeDtypeStruct((_K * _N, _N), jnp.bfloat16),
        compiler_params=pltpu.CompilerParams(
            dimension_semantics=("arbitrary",)),
    )(nbr_t)


def _layer_body(add_identity, need_stats, *refs):
    if add_identity:
        (x_ref, prm_ref, bias_ref, w_ref, S_ref, idn_ref,
         out_ref, st_ref, acc_ref, py0_ref, s_ref, ss_ref) = refs
    else:
        (x_ref, prm_ref, bias_ref, w_ref, S_ref,
         out_ref, st_ref, acc_ref, py0_ref, s_ref, ss_ref) = refs
        idn_ref = None
    bp = pl.program_id(0)
    t = pl.program_id(1)

    @pl.when((bp == 0) & (t == 0))
    def _init():
        s_ref[...] = jnp.zeros_like(s_ref)
        ss_ref[...] = jnp.zeros_like(ss_ref)

    a = prm_ref[0:1, :]
    c = prm_ref[1:2, :]
    bias = bias_ref[0:1, :]

    def _accum_stats(o, m0=None):
        if need_stats:
            orr = o.reshape(_N // 8, 8, _F)
            ps = jnp.sum(orr, axis=0)
            pss = jnp.sum(orr * orr, axis=0)
            if m0 is not None:
                ps = m0 * ps
                pss = m0 * pss
            s_ref[...] += ps
            ss_ref[...] += pss

    @pl.when(t < _T)
    def _compute():
        xs = x_ref[...]
        v = xs[:, 0].astype(jnp.float32) * a + c
        v = 0.5 * v * (1.0 + jax.lax.erf(v * _INV_SQRT2))
        vb = v.astype(jnp.bfloat16)
        vcat = jnp.concatenate([vb[b] for b in range(_P)], axis=1)
        g = jax.lax.dot_general(
            S_ref[...], vcat, (((1,), (0,)), ((), ())),
            preferred_element_type=jnp.float32).astype(jnp.bfloat16)
        m0 = (t > 0).astype(jnp.float32)
        for b in range(_P):
            gb = jnp.concatenate(
                [g[k * _N:(k + 1) * _N, b * _F:(b + 1) * _F]
                 for k in range(_K)], axis=1)
            yb = jax.lax.dot_general(
                gb, w_ref[...], (((1,), (0,)), ((), ())),
                preferred_element_type=jnp.float32) + bias
            y0 = yb[:, 0:_F]
            y1 = yb[:, _F:2 * _F]
            y2 = yb[:, 2 * _F:]
            acc = jnp.where(t > 0, acc_ref[b], 0.0)
            py0 = jnp.where(t > 0, py0_ref[b], 0.0)
            o = acc + y2
            if add_identity:
                o = o + idn_ref[b, 0]
            out_ref[b, 0] = o.astype(out_ref.dtype)
            _accum_stats(o, m0)
            acc_ref[b] = py0 + y1
            py0_ref[b] = y0

    @pl.when(t == _T)
    def _last():
        for b in range(_P):
            o = acc_ref[b]
            if add_identity:
                o = o + idn_ref[b, 0]
            out_ref[b, 0] = o.astype(out_ref.dtype)
            _accum_stats(o)

    if need_stats:
        @pl.when((bp == pl.num_programs(0) - 1) & (t == _T))
        def _st_out():
            st_ref[0:8, :] = s_ref[...]
            st_ref[8:16, :] = ss_ref[...]


def _layer(x, prm, bias, wbf, S, identity=None, need_stats=True,
           out_dtype=jnp.float32):
    add_identity = identity is not None
    body = functools.partial(_layer_body, add_identity, need_stats)
    in_specs = [
        pl.BlockSpec((_P, 1, _N, _F),
                     lambda bp, t: (bp, jnp.minimum(t, _T - 1), 0, 0)),
        pl.BlockSpec((8, _F), lambda bp, t: (0, 0)),
        pl.BlockSpec((8, _TF), lambda bp, t: (0, 0)),
        pl.BlockSpec((_K * _F, _TF), lambda bp, t: (0, 0)),
        pl.BlockSpec((_K * _N, _N), lambda bp, t: (0, 0)),
    ]
    args = [x, prm, bias, wbf, S]
    if add_identity:
        in_specs.append(
            pl.BlockSpec((_P, 1, _N, _F),
                         lambda bp, t: (bp, jnp.maximum(t - 1, 0), 0, 0)))
        args.append(identity)
    out_specs = [
        pl.BlockSpec((_P, 1, _N, _F),
                     lambda bp, t: (bp, jnp.maximum(t - 1, 0), 0, 0)),
        pl.BlockSpec((16, _F), lambda bp, t: (0, 0)),
    ]
    out_shape = [
        jax.ShapeDtypeStruct((_B, _T, _N, _F), out_dtype),
        jax.ShapeDtypeStruct((16, _F), jnp.float32),
    ]
    h, st = pl.pallas_call(
        body,
        grid=(_B // _P, _T + 1),
        in_specs=in_specs,
        out_specs=out_specs,
        out_shape=out_shape,
        scratch_shapes=[
            pltpu.VMEM((_P, _N, _F), jnp.float32),
            pltpu.VMEM((_P, _N, _F), jnp.float32),
            pltpu.VMEM((8, _F), jnp.float32),
            pltpu.VMEM((8, _F), jnp.float32),
        ],
        compiler_params=pltpu.CompilerParams(
            dimension_semantics=("arbitrary", "arbitrary")),
    )(*args)
    return h, st


def _affine(st, gamma, beta):
    cnt = float(_B * _T * _N)
    s = jnp.sum(st[0:8, :], axis=0)
    ss = jnp.sum(st[8:16, :], axis=0)
    mean = s / cnt
    var = ss / cnt - mean * mean
    rstd = jax.lax.rsqrt(var + _EPS)
    a = gamma * rstd
    c = beta - mean * a
    prm = jnp.zeros((8, _F), jnp.float32).at[0].set(a).at[1].set(c)
    return prm


def _bias_rows(b):
    return jnp.zeros((8, _TF), jnp.float32).at[0].set(b)


def kernel(x, neighbors, gamma1, beta1, W1, b1, gamma2, beta2, W2, b2):
    nbr_t = neighbors.astype(jnp.int32).T.reshape(_K, 1, _N)
    S = _onehot(nbr_t)
    st0 = _moment_sums(x.reshape(_B * _T, _N, _F))
    prm1 = _affine(st0, gamma1, beta1)
    h1, st1 = _layer(x, prm1, _bias_rows(b1 * _INV_SQRT3),
                     (W1 * _INV_SQRT3).astype(jnp.bfloat16), S,
                     identity=None, need_stats=True, out_dtype=jnp.bfloat16)
    prm2 = _affine(st1, gamma2, beta2)
    out, _ = _layer(h1, prm2, _bias_rows(b2 * _INV_SQRT3),
                    (W2 * _INV_SQRT3).astype(jnp.bfloat16), S,
                    identity=x, need_stats=False)
    return out
